# baseline (speedup 1.0000x reference)
import functools

import jax
import jax.numpy as jnp
from jax import lax
from jax.experimental import pallas as pl
from jax.experimental.pallas import tpu as pltpu
from jax.experimental.pallas import tpu_sc as plsc

_B = 16
_D = 64
_DP = 128
_HW = 1024
_K = 1024
_COMMIT = 0.25
_IMGS = 4
_C = 128

_info = plsc.get_sparse_core_info()
_NW = _info.num_cores * _info.num_subcores
_BPW = (_B * _HW) // _NW

_mesh = plsc.VectorSubcoreMesh(core_axis_name="c", subcore_axis_name="s")


@functools.partial(
    pl.kernel, mesh=_mesh,
    out_type=jax.ShapeDtypeStruct((_B * _HW, _DP), jnp.float32),
    scratch_types=[
        pltpu.VMEM((_BPW,), jnp.int32),
        pltpu.VMEM((_BPW, _DP), jnp.float32),
        pltpu.SemaphoreType.DMA,
    ],
)
def _gather_k(table_hbm, idx_hbm, out_hbm, idx_v, rows_v, sem):
    wid = lax.axis_index("s") * _info.num_cores + lax.axis_index("c")
    base = wid * _BPW
    pltpu.sync_copy(idx_hbm.at[pl.ds(base, _BPW)], idx_v)
    pltpu.async_copy(table_hbm.at[idx_v], rows_v, sem).wait()
    pltpu.sync_copy(rows_v, out_hbm.at[pl.ds(base, _BPW)])


def _vq_body(x_ref, e_ref, idx_ref, loss_ref):
    e = e_ref[...]
    esq = jnp.sum(e * e, axis=1)
    e2 = e + e
    loss = jnp.zeros((), jnp.float32)
    for k in range(_IMGS):
        xb = x_ref[k]
        xt = xb.T
        xsq = jnp.sum(xt * xt, axis=1, keepdims=True)
        mm2 = jax.lax.dot_general(xt, e2, (((1,), (1,)), ((), ())))
        xsqb = jnp.broadcast_to(xsq, (_HW, _C))
        val = (xsqb + esq[0:_C]) - mm2[:, 0:_C]
        gch = jnp.zeros((_HW, _C), jnp.int32)
        for c in range(1, _K // _C):
            d_c = (xsqb + esq[c * _C:(c + 1) * _C]) - mm2[:, c * _C:(c + 1) * _C]
            take = d_c < val
            val = jnp.where(take, d_c, val)
            gch = jnp.where(take, c, gch)
        minv = jnp.min(val, axis=1, keepdims=True)
        lane = jax.lax.broadcasted_iota(jnp.int32, (_HW, _C), 1)
        cand = jnp.where(val == minv, gch * _C + lane, _K)
        idx = jnp.min(cand, axis=1).astype(jnp.int32)
        idx_ref[k, 0, :] = idx
        loss = loss + jnp.sum(minv)
    loss_ref[...] = loss.reshape(1, 1, 1)


def kernel(x, embeddings):
    x3 = x.reshape(_B, _D, _HW)
    idx, loss = pl.pallas_call(
        _vq_body,
        grid=(_B // _IMGS,),
        in_specs=[
            pl.BlockSpec((_IMGS, _D, _HW), lambda i: (i, 0, 0)),
            pl.BlockSpec((_K, _D), lambda i: (0, 0)),
        ],
        out_specs=[
            pl.BlockSpec((_IMGS, 1, _HW), lambda i: (i, 0, 0)),
            pl.BlockSpec((1, 1, 1), lambda i: (i, 0, 0)),
        ],
        out_shape=[
            jax.ShapeDtypeStruct((_B, 1, _HW), jnp.int32),
            jax.ShapeDtypeStruct((_B // _IMGS, 1, 1), jnp.float32),
        ],
    )(x3, embeddings)
    epad = jnp.pad(embeddings, ((0, 0), (0, _DP - _D)))
    qrows = _gather_k(epad, idx.reshape(_B * _HW))
    q_t = qrows[:, :_D].reshape(_B, _HW, _D).transpose(0, 2, 1)
    out = (x3 + (q_t - x3)).reshape(x.shape)
    enc = idx.reshape(_B, _HW)
    d = jnp.sum(loss) / (_B * _D * _HW)
    total_loss = d + _COMMIT * d
    return out, total_loss, enc, embeddings


# speedup vs baseline: 1.3343x; 1.3343x over previous
import jax
import jax.numpy as jnp
from jax.experimental import pallas as pl

_B = 16
_D = 64
_HW = 1024
_K = 1024
_COMMIT = 0.25


_IMGS = 4
_C = 128


def _vq_body(x_ref, e_ref, out_ref, idx_ref, loss_ref):
    e = e_ref[...]
    esq = jnp.sum(e * e, axis=1)
    e2 = e + e
    loss = jnp.zeros((), jnp.float32)
    for k in range(_IMGS):
        xb = x_ref[k]
        xt = xb.T
        xsq = jnp.sum(xt * xt, axis=1, keepdims=True)
        mm2 = jax.lax.dot_general(xt, e2, (((1,), (1,)), ((), ())))
        xsqb = jnp.broadcast_to(xsq, (_HW, _C))
        val = (xsqb + esq[0:_C]) - mm2[:, 0:_C]
        gch = jnp.zeros((_HW, _C), jnp.int32)
        for c in range(1, _K // _C):
            d_c = (xsqb + esq[c * _C:(c + 1) * _C]) - mm2[:, c * _C:(c + 1) * _C]
            take = d_c < val
            val = jnp.where(take, d_c, val)
            gch = jnp.where(take, c, gch)
        minv = jnp.min(val, axis=1, keepdims=True)
        lane = jax.lax.broadcasted_iota(jnp.int32, (_HW, _C), 1)
        cand = jnp.where(val == minv, gch * _C + lane, _K)
        idxc = jnp.min(cand, axis=1, keepdims=True).astype(jnp.int32)
        idx_ref[k, :, :] = idxc
        iota = jax.lax.broadcasted_iota(jnp.int32, (_HW, _K), 1)
        onehot = (jnp.broadcast_to(idxc, (_HW, _K)) == iota).astype(jnp.float32)
        q_t = jax.lax.dot_general(e, onehot, (((0,), (1,)), ((), ())))
        diff = q_t - xb
        out_ref[k] = xb + diff
        loss = loss + jnp.sum(diff * diff)
    loss_ref[...] = loss.reshape(1, 1, 1)


def kernel(x, embeddings):
    x3 = x.reshape(_B, _D, _HW)
    out, idx, loss = pl.pallas_call(
        _vq_body,
        grid=(_B // _IMGS,),
        in_specs=[
            pl.BlockSpec((_IMGS, _D, _HW), lambda i: (i, 0, 0)),
            pl.BlockSpec((_K, _D), lambda i: (0, 0)),
        ],
        out_specs=[
            pl.BlockSpec((_IMGS, _D, _HW), lambda i: (i, 0, 0)),
            pl.BlockSpec((_IMGS, _HW, 1), lambda i: (i, 0, 0)),
            pl.BlockSpec((1, 1, 1), lambda i: (i, 0, 0)),
        ],
        out_shape=[
            jax.ShapeDtypeStruct((_B, _D, _HW), jnp.float32),
            jax.ShapeDtypeStruct((_B, _HW, 1), jnp.int32),
            jax.ShapeDtypeStruct((_B // _IMGS, 1, 1), jnp.float32),
        ],
    )(x3, embeddings)
    out4 = out.reshape(x.shape)
    enc = idx.reshape(_B, _HW)
    d = jnp.sum(loss) / (_B * _D * _HW)
    total_loss = d + _COMMIT * d
    return out4, total_loss, enc, embeddings


# speedup vs baseline: 1.4259x; 1.0686x over previous
import jax
import jax.numpy as jnp
from jax.experimental import pallas as pl

_B = 16
_D = 64
_HW = 1024
_K = 1024
_COMMIT = 0.25


_IMGS = 8
_C = 128


def _vq_body(x_ref, e_ref, out_ref, idx_ref, loss_ref):
    e = e_ref[...]
    esq = jnp.sum(e * e, axis=1)
    e2 = e + e
    loss = jnp.zeros((), jnp.float32)
    for k in range(_IMGS):
        xb = x_ref[k]
        xt = xb.T
        xsq = jnp.sum(xt * xt, axis=1, keepdims=True)
        mm2 = jax.lax.dot_general(xt, e2, (((1,), (1,)), ((), ())))
        xsqb = jnp.broadcast_to(xsq, (_HW, _C))
        val = (xsqb + esq[0:_C]) - mm2[:, 0:_C]
        gch = jnp.zeros((_HW, _C), jnp.int32)
        for c in range(1, _K // _C):
            d_c = (xsqb + esq[c * _C:(c + 1) * _C]) - mm2[:, c * _C:(c + 1) * _C]
            take = d_c < val
            val = jnp.where(take, d_c, val)
            gch = jnp.where(take, c, gch)
        minv = jnp.min(val, axis=1, keepdims=True)
        lane = jax.lax.broadcasted_iota(jnp.int32, (_HW, _C), 1)
        cand = jnp.where(val == minv, gch * _C + lane, _K)
        idx = jnp.min(cand, axis=1).astype(jnp.int32)
        idx_ref[k, 0, :] = idx
        iota = jax.lax.broadcasted_iota(jnp.int32, (_HW, _K), 1)
        onehot = (idx[:, None] == iota).astype(jnp.float32)
        q_t = jax.lax.dot_general(e, onehot, (((0,), (1,)), ((), ())))
        diff = q_t - xb
        out_ref[k] = xb + diff
        loss = loss + jnp.sum(diff * diff)
    loss_ref[...] = loss.reshape(1, 1, 1)


def kernel(x, embeddings):
    x3 = x.reshape(_B, _D, _HW)
    out, idx, loss = pl.pallas_call(
        _vq_body,
        grid=(_B // _IMGS,),
        in_specs=[
            pl.BlockSpec((_IMGS, _D, _HW), lambda i: (i, 0, 0)),
            pl.BlockSpec((_K, _D), lambda i: (0, 0)),
        ],
        out_specs=[
            pl.BlockSpec((_IMGS, _D, _HW), lambda i: (i, 0, 0)),
            pl.BlockSpec((_IMGS, 1, _HW), lambda i: (i, 0, 0)),
            pl.BlockSpec((1, 1, 1), lambda i: (i, 0, 0)),
        ],
        out_shape=[
            jax.ShapeDtypeStruct((_B, _D, _HW), jnp.float32),
            jax.ShapeDtypeStruct((_B, 1, _HW), jnp.int32),
            jax.ShapeDtypeStruct((_B // _IMGS, 1, 1), jnp.float32),
        ],
    )(x3, embeddings)
    out4 = out.reshape(x.shape)
    enc = idx.reshape(_B, _HW)
    d = jnp.sum(loss) / (_B * _D * _HW)
    total_loss = d + _COMMIT * d
    return out4, total_loss, enc, embeddings
